# baseline (speedup 1.0000x reference)
import jax
import jax.numpy as jnp
from jax import lax
from jax.experimental import pallas as pl
from jax.experimental.pallas import tpu as pltpu
from jax.experimental.pallas import tpu_sc as plsc

_NC = 2
_NS = 16
_NW = _NC * _NS

_CHUNK = 128
_EP = 128


def _gather_body(users_hbm, animes_hbm, uidx_hbm, aidx_hbm, u_out, a_out,
                 idx_v, rows, sem):
    nch = idx_v.shape[1]
    bpw = nch * _CHUNK
    wid = lax.axis_index("s") * _NC + lax.axis_index("c")
    base = wid * bpw
    pltpu.sync_copy(uidx_hbm.at[pl.ds(wid * nch, nch)], idx_v.at[0])
    pltpu.sync_copy(aidx_hbm.at[pl.ds(wid * nch, nch)], idx_v.at[1])
    cus = [pltpu.async_copy(users_hbm.at[idx_v.at[0, j]],
                            rows.at[pl.ds(j * _CHUNK, _CHUNK)], sem)
           for j in range(nch)]
    for c in cus:
        c.wait()
    pltpu.sync_copy(rows, u_out.at[pl.ds(base, bpw)])
    cas = [pltpu.async_copy(animes_hbm.at[idx_v.at[1, j]],
                            rows.at[pl.ds(j * _CHUNK, _CHUNK)], sem)
           for j in range(nch)]
    for c in cas:
        c.wait()
    pltpu.sync_copy(rows, a_out.at[pl.ds(base, bpw)])


def _sc_gather(users_p, animes_p, uidx, aidx):
    B = uidx.shape[0]
    bpw = B // _NW
    nch = bpw // _CHUNK
    mesh = plsc.VectorSubcoreMesh(core_axis_name="c", subcore_axis_name="s")
    return pl.kernel(
        _gather_body,
        out_type=(jax.ShapeDtypeStruct((B, _EP), jnp.float32),
                  jax.ShapeDtypeStruct((B, _EP), jnp.float32)),
        mesh=mesh,
        scratch_types=[
            pltpu.VMEM((2, nch, _CHUNK), jnp.int32),
            pltpu.VMEM((bpw, _EP), jnp.float32),
            pltpu.SemaphoreType.DMA,
        ],
    )(users_p, animes_p, uidx.reshape(B // _CHUNK, _CHUNK),
      aidx.reshape(B // _CHUNK, _CHUNK))


def _mlp_body(u_ref, a_ref, x_ref, w1u_ref, w1a_ref, w1x_ref, b1_ref,
              w2_ref, b2_ref, w3_ref, b3_ref, o_ref):
    h = jnp.dot(u_ref[...], w1u_ref[...], preferred_element_type=jnp.float32)
    h = h + jnp.dot(a_ref[...], w1a_ref[...], preferred_element_type=jnp.float32)
    h = h + jnp.dot(x_ref[...], w1x_ref[...], preferred_element_type=jnp.float32)
    h = jnp.maximum(h + b1_ref[...], 0.0)
    h = jnp.dot(h, w2_ref[...], preferred_element_type=jnp.float32)
    h = jnp.maximum(h + b2_ref[...], 0.0)
    o = jnp.dot(h, w3_ref[...], preferred_element_type=jnp.float32)
    o_ref[...] = jax.nn.sigmoid(o + b3_ref[...])


def _tc_mlp(u, a, x, w1u, w1a, w1x, b1, w2, b2, w3, b3, blk):
    B = u.shape[0]
    F = x.shape[1]
    H1 = w1u.shape[1]
    H2 = w2.shape[1]
    grid = B // blk
    row = lambda i: (i, 0)
    fixed = lambda i: (0, 0)
    return pl.pallas_call(
        _mlp_body,
        grid=(grid,),
        in_specs=[
            pl.BlockSpec((blk, _EP), row),
            pl.BlockSpec((blk, _EP), row),
            pl.BlockSpec((blk, F), row),
            pl.BlockSpec((_EP, H1), fixed),
            pl.BlockSpec((_EP, H1), fixed),
            pl.BlockSpec((F, H1), fixed),
            pl.BlockSpec((1, H1), fixed),
            pl.BlockSpec((H1, H2), fixed),
            pl.BlockSpec((1, H2), fixed),
            pl.BlockSpec((H2, 1), fixed),
            pl.BlockSpec((1, 1), fixed),
        ],
        out_specs=pl.BlockSpec((blk, 1), row),
        out_shape=jax.ShapeDtypeStruct((B, 1), jnp.float32),
    )(u, a, x, w1u, w1a, w1x, b1, w2, b2, w3, b3)


def kernel(x, users, animes, W1, b1, W2, b2, W3, b3):
    E = users.shape[1]
    H1 = W1.shape[0]
    uidx = x[:, 0].astype(jnp.int32)
    aidx = x[:, 1].astype(jnp.int32)
    users_p = jnp.pad(users, ((0, 0), (0, _EP - E)))
    animes_p = jnp.pad(animes, ((0, 0), (0, _EP - E)))
    u, a = _sc_gather(users_p, animes_p, uidx, aidx)

    w1t = W1.T
    w1u = jnp.pad(w1t[:E], ((0, _EP - E), (0, 0)))
    w1a = jnp.pad(w1t[E:2 * E], ((0, _EP - E), (0, 0)))
    w1x = jnp.concatenate([jnp.zeros((2, H1), jnp.float32), w1t[2 * E:]], axis=0)
    return _tc_mlp(u, a, x, w1u, w1a, w1x, b1.reshape(1, -1),
                   W2.T, b2.reshape(1, -1), W3.T, b3.reshape(1, -1), blk=2048)


# speedup vs baseline: 1.4385x; 1.4385x over previous
import jax
import jax.numpy as jnp
from jax import lax
from jax.experimental import pallas as pl
from jax.experimental.pallas import tpu as pltpu
from jax.experimental.pallas import tpu_sc as plsc

_NC = 2
_NS = 16
_NW = _NC * _NS

_CHUNK = 128
_H1 = 128


def _tx_body(tab_ref, w_ref, o_ref):
    o_ref[...] = jnp.dot(tab_ref[...], w_ref[...],
                         preferred_element_type=jnp.float32)


def _tx(tab, w, blk):
    V, E = tab.shape
    H = w.shape[1]
    return pl.pallas_call(
        _tx_body,
        grid=(pl.cdiv(V, blk),),
        in_specs=[pl.BlockSpec((blk, E), lambda i: (i, 0)),
                  pl.BlockSpec((E, H), lambda i: (0, 0))],
        out_specs=pl.BlockSpec((blk, H), lambda i: (i, 0)),
        out_shape=jax.ShapeDtypeStruct((V, H), jnp.float32),
    )(tab, w)


def _gather_body(users_hbm, animes_hbm, uidx_hbm, aidx_hbm, u_out, a_out,
                 idx_v, rows, sem):
    nch = idx_v.shape[1]
    bpw = nch * _CHUNK
    wid = lax.axis_index("s") * _NC + lax.axis_index("c")
    base = wid * bpw
    pltpu.sync_copy(uidx_hbm.at[pl.ds(wid * nch, nch)], idx_v.at[0])
    pltpu.sync_copy(aidx_hbm.at[pl.ds(wid * nch, nch)], idx_v.at[1])
    cus = [pltpu.async_copy(users_hbm.at[idx_v.at[0, j]],
                            rows.at[pl.ds(j * _CHUNK, _CHUNK)], sem)
           for j in range(nch)]
    for c in cus:
        c.wait()
    pltpu.sync_copy(rows, u_out.at[pl.ds(base, bpw)])
    cas = [pltpu.async_copy(animes_hbm.at[idx_v.at[1, j]],
                            rows.at[pl.ds(j * _CHUNK, _CHUNK)], sem)
           for j in range(nch)]
    for c in cas:
        c.wait()
    pltpu.sync_copy(rows, a_out.at[pl.ds(base, bpw)])


def _sc_gather(users_t, animes_t, uidx, aidx):
    B = uidx.shape[0]
    bpw = B // _NW
    nch = bpw // _CHUNK
    mesh = plsc.VectorSubcoreMesh(core_axis_name="c", subcore_axis_name="s")
    return pl.kernel(
        _gather_body,
        out_type=(jax.ShapeDtypeStruct((B, _H1), jnp.float32),
                  jax.ShapeDtypeStruct((B, _H1), jnp.float32)),
        mesh=mesh,
        scratch_types=[
            pltpu.VMEM((2, nch, _CHUNK), jnp.int32),
            pltpu.VMEM((bpw, _H1), jnp.float32),
            pltpu.SemaphoreType.DMA,
        ],
    )(users_t, animes_t, uidx.reshape(B // _CHUNK, _CHUNK),
      aidx.reshape(B // _CHUNK, _CHUNK))


def _mlp_body(u_ref, a_ref, x_ref, w1x_ref, b1_ref,
              w2_ref, b2_ref, w3_ref, b3_ref, o_ref):
    h = u_ref[...] + a_ref[...] + b1_ref[...]
    h = h + jnp.dot(x_ref[...], w1x_ref[...], preferred_element_type=jnp.float32)
    h = jnp.maximum(h, 0.0)
    h = jnp.dot(h, w2_ref[...], preferred_element_type=jnp.float32)
    h = jnp.maximum(h + b2_ref[...], 0.0)
    o = jnp.dot(h, w3_ref[...], preferred_element_type=jnp.float32)
    o_ref[...] = jax.nn.sigmoid(o + b3_ref[...])


def _tc_mlp(u, a, x, w1x, b1, w2, b2, w3, b3, blk):
    B = u.shape[0]
    F = x.shape[1]
    H2 = w2.shape[1]
    row = lambda i: (i, 0)
    fixed = lambda i: (0, 0)
    return pl.pallas_call(
        _mlp_body,
        grid=(B // blk,),
        in_specs=[
            pl.BlockSpec((blk, _H1), row),
            pl.BlockSpec((blk, _H1), row),
            pl.BlockSpec((blk, F), row),
            pl.BlockSpec((F, _H1), fixed),
            pl.BlockSpec((1, _H1), fixed),
            pl.BlockSpec((_H1, H2), fixed),
            pl.BlockSpec((1, H2), fixed),
            pl.BlockSpec((H2, 1), fixed),
            pl.BlockSpec((1, 1), fixed),
        ],
        out_specs=pl.BlockSpec((blk, 1), row),
        out_shape=jax.ShapeDtypeStruct((B, 1), jnp.float32),
    )(u, a, x, w1x, b1, w2, b2, w3, b3)


def kernel(x, users, animes, W1, b1, W2, b2, W3, b3):
    E = users.shape[1]
    H1 = W1.shape[0]
    uidx = x[:, 0].astype(jnp.int32)
    aidx = x[:, 1].astype(jnp.int32)

    w1t = W1.T
    users_t = _tx(users, w1t[:E], blk=2048)
    animes_t = _tx(animes, w1t[E:2 * E], blk=2048)
    u, a = _sc_gather(users_t, animes_t, uidx, aidx)

    w1x = jnp.concatenate([jnp.zeros((2, H1), jnp.float32), w1t[2 * E:]], axis=0)
    return _tc_mlp(u, a, x, w1x, b1.reshape(1, -1),
                   W2.T, b2.reshape(1, -1), W3.T, b3.reshape(1, -1), blk=2048)
